# Initial kernel scaffold; baseline (speedup 1.0000x reference)
import jax
import jax.numpy as jnp
from jax.experimental import pallas as pl


def kernel(x0, edge_index0, edge_attr0, batch0, x1, edge_index1, edge_attr1, batch1, enc_W, enc_b, enc_E, p0_W1, p0_b1, p0_W2, p0_b2, p1_W1, p1_b1, p1_W2, p1_b2):
    raise NotImplementedError("write your pallas kernel here")


# speedup vs baseline: 4.1422x; 4.1422x over previous
import functools

import jax
import jax.numpy as jnp
from jax import lax
from jax.experimental import pallas as pl
from jax.experimental.pallas import tpu as pltpu
from jax.experimental.pallas import tpu_sc as plsc

N = 10000
E = 160000
D = 300
DE = 16
B = 256
L = 5
TEMP = 0.04

DP = 320
H = DP // 2
NSUB = 16
EPS = E // NSUB
KB = 80
NB = EPS // KB
N_PAD = 10240
RPS = N_PAD // NSUB

BM = 400
GRID_N = N // BM

_mesh = plsc.VectorSubcoreMesh(core_axis_name="c", subcore_axis_name="s")


@functools.partial(
    pl.kernel,
    mesh=_mesh,
    out_type=[
        jax.ShapeDtypeStruct((N_PAD, H), jnp.float32),
        jax.ShapeDtypeStruct((N_PAD, H), jnp.float32),
    ],
    scratch_types=[
        pltpu.VMEM((KB,), jnp.int32),
        pltpu.VMEM((KB,), jnp.int32),
        pltpu.VMEM((KB, H), jnp.float32),
        pltpu.VMEM_SHARED((N_PAD, H), jnp.float32),
        pltpu.SemaphoreType.DMA,
    ],
    compiler_params=pltpu.CompilerParams(use_tc_tiling_on_sc=False),
)
def _sc_neighbor_sum(hlo_hbm, hhi_hbm, src_hbm, dst_hbm, z_hbm,
                     olo_hbm, ohi_hbm, sidx, didx, rows, acc, sem):
    cid = lax.axis_index("c")
    sid = lax.axis_index("s")
    slab = pl.ds(sid * RPS, RPS)
    pltpu.sync_copy(z_hbm.at[slab], acc.at[slab])
    plsc.subcore_barrier()

    def run(tab, out):
        def body(i, carry):
            base = sid * EPS + i * KB
            pltpu.sync_copy(src_hbm.at[pl.ds(base, KB)], sidx)
            pltpu.sync_copy(dst_hbm.at[pl.ds(base, KB)], didx)
            pltpu.async_copy(tab.at[sidx], rows, sem).wait()
            pltpu.sync_copy(rows, acc.at[didx], add=True)
            return carry

        lax.fori_loop(0, NB, body, 0)
        plsc.subcore_barrier()
        pltpu.sync_copy(acc.at[slab], out.at[slab])

    @pl.when(cid == 0)
    def _():
        run(hlo_hbm, olo_hbm)

    @pl.when(cid == 1)
    def _():
        run(hhi_hbm, ohi_hbm)


@functools.partial(
    pl.kernel,
    mesh=_mesh,
    out_type=[
        jax.ShapeDtypeStruct((N_PAD, DE), jnp.float32),
        jax.ShapeDtypeStruct((N_PAD, DE), jnp.float32),
    ],
    scratch_types=[
        pltpu.VMEM((KB,), jnp.int32),
        pltpu.VMEM((KB, DE), jnp.float32),
        pltpu.VMEM_SHARED((N_PAD, DE), jnp.float32),
    ],
    compiler_params=pltpu.CompilerParams(use_tc_tiling_on_sc=False),
)
def _sc_edge_attr_sum(ea0_hbm, dst0_hbm, ea1_hbm, dst1_hbm, z_hbm,
                      o0_hbm, o1_hbm, didx, eav, acc):
    cid = lax.axis_index("c")
    sid = lax.axis_index("s")
    slab = pl.ds(sid * RPS, RPS)
    pltpu.sync_copy(z_hbm.at[slab], acc.at[slab])
    plsc.subcore_barrier()

    def run(ea, dstr, out):
        def body(i, carry):
            base = sid * EPS + i * KB
            pltpu.sync_copy(ea.at[pl.ds(base, KB)], eav)
            pltpu.sync_copy(dstr.at[pl.ds(base, KB)], didx)
            pltpu.sync_copy(eav, acc.at[didx], add=True)
            return carry

        lax.fori_loop(0, NB, body, 0)
        plsc.subcore_barrier()
        pltpu.sync_copy(acc.at[slab], out.at[slab])

    @pl.when(cid == 0)
    def _():
        run(ea0_hbm, dst0_hbm, o0_hbm)

    @pl.when(cid == 1)
    def _():
        run(ea1_hbm, dst1_hbm, o1_hbm)


def _tc_layer_body(relu, lo_ref, hi_ref, ea_ref, ep_ref, w0_ref, w1_ref,
                   b_ref, olo_ref, ohi_ref):
    t = jnp.dot(ea_ref[...], ep_ref[...], preferred_element_type=jnp.float32, precision=lax.Precision.HIGHEST)
    mlo = lo_ref[...] + t[:, :H]
    mhi = hi_ref[...] + t[:, H:]
    out = (jnp.dot(mlo, w0_ref[...], preferred_element_type=jnp.float32, precision=lax.Precision.DEFAULT)
           + jnp.dot(mhi, w1_ref[...], preferred_element_type=jnp.float32, precision=lax.Precision.DEFAULT)
           + b_ref[...])
    if relu:
        out = jnp.maximum(out, 0.0)
    olo_ref[...] = out[:, :H]
    ohi_ref[...] = out[:, H:]


_LAYER_IN_SPECS = [
    pl.BlockSpec((BM, H), lambda i: (i, 0)),
    pl.BlockSpec((BM, H), lambda i: (i, 0)),
    pl.BlockSpec((BM, DE), lambda i: (i, 0)),
    pl.BlockSpec((DE, DP), lambda i: (0, 0)),
    pl.BlockSpec((H, DP), lambda i: (0, 0)),
    pl.BlockSpec((H, DP), lambda i: (0, 0)),
    pl.BlockSpec((1, DP), lambda i: (0, 0)),
]

_tc_layer = pl.pallas_call(
    functools.partial(_tc_layer_body, True),
    grid=(GRID_N,),
    in_specs=_LAYER_IN_SPECS,
    out_specs=[pl.BlockSpec((BM, H), lambda i: (i, 0)),
               pl.BlockSpec((BM, H), lambda i: (i, 0))],
    out_shape=[jax.ShapeDtypeStruct((N, H), jnp.float32)] * 2,
)


def _tc_final_body(lo_ref, hi_ref, ea_ref, ep_ref, w0_ref, w1_ref, b_ref,
                   bat_ref, pool_ref):
    t = jnp.dot(ea_ref[...], ep_ref[...], preferred_element_type=jnp.float32, precision=lax.Precision.HIGHEST)
    mlo = lo_ref[...] + t[:, :H]
    mhi = hi_ref[...] + t[:, H:]
    out = (jnp.dot(mlo, w0_ref[...], preferred_element_type=jnp.float32, precision=lax.Precision.DEFAULT)
           + jnp.dot(mhi, w1_ref[...], preferred_element_type=jnp.float32, precision=lax.Precision.DEFAULT)
           + b_ref[...])
    col = lax.broadcasted_iota(jnp.int32, (BM, DP), 1)
    out = jnp.where(col == D, 1.0, out)
    ids = bat_ref[0, 0, :]
    seg = lax.broadcasted_iota(jnp.int32, (BM, B), 1)
    onehot = jnp.where(seg == ids[:, None], 1.0, 0.0)
    part = lax.dot_general(onehot, out, (((0,), (0,)), ((), ())),
                           preferred_element_type=jnp.float32, precision=lax.Precision.HIGHEST)

    @pl.when(pl.program_id(0) == 0)
    def _():
        pool_ref[...] = jnp.zeros_like(pool_ref)

    pool_ref[...] += part


_tc_final_pool = pl.pallas_call(
    _tc_final_body,
    grid=(GRID_N,),
    in_specs=_LAYER_IN_SPECS + [pl.BlockSpec((1, 1, BM), lambda i: (i, 0, 0))],
    out_specs=pl.BlockSpec((B, DP), lambda i: (0, 0)),
    out_shape=jax.ShapeDtypeStruct((B, DP), jnp.float32),
)


def _tc_project_body(p0_ref, p1_ref, w1a_ref, b1a_ref, w2a_ref, b2a_ref,
                     w1b_ref, b1b_ref, w2b_ref, b2b_ref, out_ref):
    def proj(p, w1, b1, w2, b2):
        cnt = jnp.maximum(p[:, D:D + 1], 1.0)
        mean = p / cnt
        z = jnp.maximum(
            jnp.dot(mean, w1, preferred_element_type=jnp.float32, precision=lax.Precision.DEFAULT) + b1, 0.0)
        o = jnp.dot(z, w2, preferred_element_type=jnp.float32, precision=lax.Precision.DEFAULT) + b2
        n = jnp.sqrt(jnp.sum(o * o, axis=1, keepdims=True))
        return o / jnp.maximum(n, 1e-12)

    f0 = proj(p0_ref[...], w1a_ref[...], b1a_ref[...], w2a_ref[...],
              b2a_ref[...])
    f1 = proj(p1_ref[...], w1b_ref[...], b1b_ref[...], w2b_ref[...],
              b2b_ref[...])
    out_ref[...] = lax.dot_general(f0, f1, (((1,), (1,)), ((), ())),
                                   preferred_element_type=jnp.float32, precision=lax.Precision.DEFAULT) * (1.0 / TEMP)


_tc_project = pl.pallas_call(
    _tc_project_body,
    out_shape=jax.ShapeDtypeStruct((B, B), jnp.float32),
)


def kernel(x0, edge_index0, edge_attr0, batch0, x1, edge_index1, edge_attr1,
           batch1, enc_W, enc_b, enc_E, p0_W1, p0_b1, p0_W2, p0_b2, p1_W1,
           p1_b1, p1_W2, p1_b2):
    f32 = jnp.float32
    Wp = jnp.zeros((L, DP, DP), f32).at[:, :D, :D].set(enc_W)
    Ep = jnp.zeros((L, DE, DP), f32).at[:, :, :D].set(enc_E)
    bp = jnp.zeros((L, 1, DP), f32).at[:, 0, :D].set(enc_b)
    zeros_h = jnp.zeros((N_PAD, H), f32)
    zeros16 = jnp.zeros((N_PAD, DE), f32)

    def pad_w(w):
        return jnp.zeros((DP, DP), f32).at[:D, :D].set(w)

    def pad_b(b):
        return jnp.zeros((1, DP), f32).at[0, :D].set(b)

    eagg0, eagg1 = _sc_edge_attr_sum(edge_attr0, edge_index0[1],
                                     edge_attr1, edge_index1[1], zeros16)

    def encode(x, src, dst, eagg, batch3d):
        h_lo = x[:, :H]
        h_hi = jnp.pad(x[:, H:], ((0, 0), (0, DP - D)))
        for l in range(L - 1):
            a_lo, a_hi = _sc_neighbor_sum(h_lo, h_hi, src, dst, zeros_h)
            h_lo, h_hi = _tc_layer(a_lo, a_hi, eagg, Ep[l], Wp[l, :H],
                                   Wp[l, H:], bp[l])
        a_lo, a_hi = _sc_neighbor_sum(h_lo, h_hi, src, dst, zeros_h)
        return _tc_final_pool(a_lo, a_hi, eagg, Ep[L - 1], Wp[L - 1, :H],
                              Wp[L - 1, H:], bp[L - 1], batch3d)

    pooled0 = encode(x0, edge_index0[0], edge_index0[1], eagg0,
                     batch0.reshape(GRID_N, 1, BM))
    pooled1 = encode(x1, edge_index1[0], edge_index1[1], eagg1,
                     batch1.reshape(GRID_N, 1, BM))

    logits = _tc_project(pooled0, pooled1,
                         pad_w(p0_W1), pad_b(p0_b1), pad_w(p0_W2), pad_b(p0_b2),
                         pad_w(p1_W1), pad_b(p1_b1), pad_w(p1_W2), pad_b(p1_b2))
    labels = jnp.arange(B, dtype=jnp.int32)
    return (logits, labels)
